# baseline (speedup 1.0000x reference)
import functools

import jax
import jax.numpy as jnp
from jax import lax
from jax.experimental import pallas as pl
from jax.experimental.pallas import tpu as pltpu
from jax.experimental.pallas import tpu_sc as plsc

N = 100000
E = 3200000
D = 16

NC = 2
NS = 16
NW = NC * NS

EPT = 102400
ROWS_PER_TILE = EPT // 128
E_PAD = EPT * NW
REAL_ROWS = E // 128
PAD_ROWS = E_PAD // 128 - REAL_ROWS

NPAD = 100352
RPT = NPAD // NS
PAD_SINK_ROWS = 352


def _sc_aggregate(with_cnt: bool, CH: int):
    NITER = ROWS_PER_TILE // CH
    NQ = NITER // 4
    mesh = plsc.VectorSubcoreMesh(core_axis_name="c", subcore_axis_name="s")
    out_type = [jax.ShapeDtypeStruct((NC, NPAD, D), jnp.float32)]
    if with_cnt:
        out_type.append(jax.ShapeDtypeStruct((NC, NPAD), jnp.float32))
    scratch = [
        pltpu.VMEM_SHARED((NPAD, D), jnp.float32),
    ]
    scratch += [pltpu.VMEM((CH, 128), jnp.int32) for _ in range(8)]
    scratch += [pltpu.VMEM((CH * 128, D), jnp.float32) for _ in range(2)]
    scratch += [pltpu.SemaphoreType.DMA for _ in range(4)]
    scratch += [pltpu.SemaphoreType.DMA for _ in range(2)]
    scratch += [pltpu.SemaphoreType.DMA for _ in range(2)]
    if with_cnt:
        scratch += [
            pltpu.VMEM_SHARED((NPAD,), jnp.float32),
            pltpu.VMEM((CH * 128,), jnp.float32),
            pltpu.SemaphoreType.DMA,
            pltpu.SemaphoreType.DMA,
        ]

    def body(x_hbm, srcR, dstR, srcP, dstP, z16_hbm, z1_hbm, *rest):
        if with_cnt:
            (agg_out, cnt_out, agg_sp,
             sv0, sv1, sv2, sv3, dv0, dv1, dv2, dv3, rv0, rv1,
             is0, is1, is2, is3, gs0, gs1, ss0, ss1,
             cnt_sp, ones_v, cs0, cs1) = rest
        else:
            (agg_out, agg_sp,
             sv0, sv1, sv2, sv3, dv0, dv1, dv2, dv3, rv0, rv1,
             is0, is1, is2, is3, gs0, gs1, ss0, ss1) = rest
        sv = [sv0, sv1, sv2, sv3]
        dv = [dv0, dv1, dv2, dv3]
        rv = [rv0, rv1]
        isem = [is0, is1, is2, is3]
        gsem = [gs0, gs1]
        ssem = [ss0, ss1]
        if with_cnt:
            csem = [cs0, cs1]
        c = lax.axis_index("c")
        s = lax.axis_index("s")
        wid = s * NC + c

        pltpu.sync_copy(z16_hbm.at[pl.ds(s * RPT, RPT)],
                        agg_sp.at[pl.ds(s * RPT, RPT)])
        if with_cnt:
            pltpu.sync_copy(z1_hbm.at[pl.ds(s * RPT, RPT)],
                            cnt_sp.at[pl.ds(s * RPT, RPT)])
            for i in range(CH * 8):
                ones_v[pl.ds(i * 16, 16)] = jnp.full((16,), 1.0, jnp.float32)
        plsc.subcore_barrier()

        row0 = wid * ROWS_PER_TILE

        def fire_idx(r, k):
            @pl.when(r < REAL_ROWS)
            def _():
                pltpu.async_copy(srcR.at[pl.ds(r, CH)], sv[k], isem[k])
                pltpu.async_copy(dstR.at[pl.ds(r, CH)], dv[k], isem[k])

            @pl.when(r >= REAL_ROWS)
            def _():
                rp = r - REAL_ROWS
                pltpu.async_copy(srcP.at[pl.ds(rp, CH)], sv[k], isem[k])
                pltpu.async_copy(dstP.at[pl.ds(rp, CH)], dv[k], isem[k])

        def wait_idx(k):
            pltpu.make_async_copy(srcR.at[pl.ds(0, CH)], sv[k],
                                  isem[k]).wait()
            pltpu.make_async_copy(dstR.at[pl.ds(0, CH)], dv[k],
                                  isem[k]).wait()

        def fire_gathers(k, m):
            for j in range(CH):
                pltpu.async_copy(x_hbm.at[sv[k].at[j]],
                                 rv[m].at[pl.ds(j * 128, 128)], gsem[m])

        def wait_gathers(m):
            pltpu.make_async_copy(x_hbm.at[pl.ds(0, CH * 128)], rv[m],
                                  gsem[m]).wait()

        def fire_scatters(k, m):
            for j in range(CH):
                pltpu.async_copy(rv[m].at[pl.ds(j * 128, 128)],
                                 agg_sp.at[dv[k].at[j]], ssem[m], add=True)
            if with_cnt:
                for j in range(CH):
                    pltpu.async_copy(ones_v.at[pl.ds(j * 128, 128)],
                                     cnt_sp.at[dv[k].at[j]], csem[m],
                                     add=True)

        def wait_scatters(m):
            pltpu.make_async_copy(rv[m], agg_sp.at[pl.ds(0, CH * 128)],
                                  ssem[m]).wait()
            if with_cnt:
                pltpu.make_async_copy(ones_v, cnt_sp.at[pl.ds(0, CH * 128)],
                                      csem[m]).wait()

        pltpu.sync_copy(srcR.at[pl.ds(row0, CH)], sv[0])
        pltpu.sync_copy(dstR.at[pl.ds(row0, CH)], dv[0])
        fire_gathers(0, 0)
        pltpu.async_copy(srcR.at[pl.ds(row0 + CH, CH)], sv[1], isem[1])
        pltpu.async_copy(dstR.at[pl.ds(row0 + CH, CH)], dv[1], isem[1])

        def quad(q, carry):
            r0 = row0 + 4 * q * CH
            for k in range(4):
                m = k % 2
                wait_gathers(m)

                if k < 2:
                    fire_idx(r0 + (k + 2) * CH, (k + 2) % 4)
                else:
                    @pl.when(q < NQ - 1)
                    def _(k=k, r0=r0):
                        fire_idx(r0 + (k + 2) * CH, (k + 2) % 4)

                if k == 0:
                    @pl.when(q > 0)
                    def _():
                        wait_scatters(1)
                else:
                    wait_scatters((k - 1) % 2)

                fire_scatters(k, m)

                if k < 3:
                    wait_idx(k + 1)
                    fire_gathers(k + 1, (k + 1) % 2)
                else:
                    @pl.when(q < NQ - 1)
                    def _():
                        wait_idx(0)
                        fire_gathers(0, 0)
            return carry

        lax.fori_loop(0, NQ, quad, 0)
        wait_scatters(1)
        plsc.subcore_barrier()

        pltpu.sync_copy(agg_sp.at[pl.ds(s * RPT, RPT)],
                        agg_out.at[c, pl.ds(s * RPT, RPT)])
        if with_cnt:
            pltpu.sync_copy(cnt_sp.at[pl.ds(s * RPT, RPT)],
                            cnt_out.at[c, pl.ds(s * RPT, RPT)])

    return pl.kernel(body, out_type=tuple(out_type), mesh=mesh,
                     scratch_types=scratch,
                     compiler_params=pltpu.CompilerParams(
                         use_tc_tiling_on_sc=False))


_sc_pass1 = _sc_aggregate(with_cnt=True, CH=4)
_sc_pass2 = _sc_aggregate(with_cnt=False, CH=5)


NP8 = NPAD // 8
NR = N // 8
BLKC = 1792
BLKR = 1792


def _sage_body(agg_a, agg_b, cnt_a, cnt_b, rep, x, wl, bl, wr, o):
    inv = 1.0 / jnp.maximum(cnt_a[0] + cnt_b[0], 1.0)
    inv16 = jnp.dot(inv, rep[...], preferred_element_type=jnp.float32)
    mean = (agg_a[0] + agg_b[0]) * inv16
    y = (jnp.dot(mean, wl[...], preferred_element_type=jnp.float32)
         + jnp.dot(x[...], wr[...], preferred_element_type=jnp.float32)
         + bl[...])
    o[...] = jnp.maximum(y, 0.0)


_sage_tc = pl.pallas_call(
    _sage_body,
    grid=(NP8 // BLKR,),
    in_specs=[pl.BlockSpec((1, BLKR, 128), lambda i: (0, i, 0)),
              pl.BlockSpec((1, BLKR, 128), lambda i: (1, i, 0)),
              pl.BlockSpec((1, BLKR, 8), lambda i: (0, i, 0)),
              pl.BlockSpec((1, BLKR, 8), lambda i: (1, i, 0)),
              pl.BlockSpec((8, 128), lambda i: (0, 0)),
              pl.BlockSpec((BLKR, 128), lambda i: (i, 0)),
              pl.BlockSpec((128, 128), lambda i: (0, 0)),
              pl.BlockSpec((1, 128), lambda i: (0, 0)),
              pl.BlockSpec((128, 128), lambda i: (0, 0))],
    out_specs=pl.BlockSpec((BLKR, 128), lambda i: (i, 0)),
    out_shape=jax.ShapeDtypeStruct((NR, 128), jnp.float32),
)


def _final_body(agg_a, agg_b, cnt_a, cnt_b, rep, x1, x0,
                wl, bl, wr, w3a, w3b, w3c, b3, o):
    inv = 1.0 / jnp.maximum(cnt_a[0] + cnt_b[0], 1.0)
    inv16 = jnp.dot(inv, rep[...], preferred_element_type=jnp.float32)
    mean = (agg_a[0] + agg_b[0]) * inv16
    y = (jnp.dot(mean, wl[...], preferred_element_type=jnp.float32)
         + jnp.dot(x1[...], wr[...], preferred_element_type=jnp.float32)
         + bl[...])
    x2 = jnp.maximum(y, 0.0)
    o[...] = (jnp.dot(x0[...], w3a[...], preferred_element_type=jnp.float32)
              + jnp.dot(x1[...], w3b[...], preferred_element_type=jnp.float32)
              + jnp.dot(x2, w3c[...], preferred_element_type=jnp.float32)
              + b3[...])


_final_tc = pl.pallas_call(
    _final_body,
    grid=(NP8 // BLKR,),
    in_specs=[pl.BlockSpec((1, BLKR, 128), lambda i: (0, i, 0)),
              pl.BlockSpec((1, BLKR, 128), lambda i: (1, i, 0)),
              pl.BlockSpec((1, BLKR, 8), lambda i: (0, i, 0)),
              pl.BlockSpec((1, BLKR, 8), lambda i: (1, i, 0)),
              pl.BlockSpec((8, 128), lambda i: (0, 0)),
              pl.BlockSpec((BLKR, 128), lambda i: (i, 0)),
              pl.BlockSpec((BLKR, 128), lambda i: (i, 0)),
              pl.BlockSpec((128, 128), lambda i: (0, 0)),
              pl.BlockSpec((1, 128), lambda i: (0, 0)),
              pl.BlockSpec((128, 128), lambda i: (0, 0)),
              pl.BlockSpec((128, 256), lambda i: (0, 0)),
              pl.BlockSpec((128, 256), lambda i: (0, 0)),
              pl.BlockSpec((128, 256), lambda i: (0, 0)),
              pl.BlockSpec((1, 256), lambda i: (0, 0))],
    out_specs=pl.BlockSpec((BLKR, 256), lambda i: (i, 0)),
    out_shape=jax.ShapeDtypeStruct((NR, 256), jnp.float32),
)


DET_W = 128000


def _detile_body(e, o_src, o_dst):
    o_src[...] = e[0].reshape(DET_W // 128, 128)
    o_dst[...] = e[1].reshape(DET_W // 128, 128)


_detile_tc = pl.pallas_call(
    _detile_body,
    grid=(E // DET_W,),
    in_specs=[pl.BlockSpec((2, DET_W), lambda i: (0, i))],
    out_specs=[pl.BlockSpec((DET_W // 128, 128), lambda i: (i, 0)),
               pl.BlockSpec((DET_W // 128, 128), lambda i: (i, 0))],
    out_shape=[jax.ShapeDtypeStruct((REAL_ROWS, 128), jnp.int32),
               jax.ShapeDtypeStruct((REAL_ROWS, 128), jnp.int32)],
)


def kernel(x0, edge_index, Wl1, bl1, Wr1, Wl2, bl2, Wr2, W3, b3):
    npad_e = E_PAD - E
    pad_idx = jnp.arange(npad_e, dtype=jnp.int32)
    pad_src = ((pad_idx * 7919) % N).reshape(PAD_ROWS, 128)
    pad_dst = (N + (pad_idx % PAD_SINK_ROWS)).reshape(PAD_ROWS, 128)
    src2d, dst2d = _detile_tc(edge_index.astype(jnp.int32))

    z16 = jnp.zeros((NPAD, D), jnp.float32)
    z1 = jnp.zeros((NPAD,), jnp.float32)

    eye8 = jnp.eye(8, dtype=jnp.float32)
    rep = jnp.kron(eye8, jnp.ones((1, 16), jnp.float32))
    wl1bd = jnp.kron(eye8, Wl1)
    wr1bd = jnp.kron(eye8, Wr1)
    wl2bd = jnp.kron(eye8, Wl2)
    wr2bd = jnp.kron(eye8, Wr2)
    w3abd = jnp.kron(eye8, W3[0:16])
    w3bbd = jnp.kron(eye8, W3[16:32])
    w3cbd = jnp.kron(eye8, W3[32:48])
    bl1t = jnp.tile(bl1, 8).reshape(1, 128)
    bl2t = jnp.tile(bl2, 8).reshape(1, 128)
    b3t = jnp.tile(b3, 8).reshape(1, 256)

    agg1, cnt = _sc_pass1(x0, src2d, dst2d, pad_src, pad_dst, z16, z1)
    cntp = cnt.reshape(NC, NP8, 8)

    x0p = x0.reshape(NR, 128)
    x1p = _sage_tc(agg1.reshape(NC, NP8, 128), agg1.reshape(NC, NP8, 128),
                   cntp, cntp, rep, x0p, wl1bd, bl1t, wr1bd)

    (agg2,) = _sc_pass2(x1p.reshape(N, D), src2d, dst2d, pad_src, pad_dst, z16, z1)

    outp = _final_tc(agg2.reshape(NC, NP8, 128), agg2.reshape(NC, NP8, 128),
                     cntp, cntp, rep, x1p, x0p, wl2bd, bl2t, wr2bd,
                     w3abd, w3bbd, w3cbd, b3t)
    return outp.reshape(N, 32)


# speedup vs baseline: 1.0084x; 1.0084x over previous
import functools

import jax
import jax.numpy as jnp
from jax import lax
from jax.experimental import pallas as pl
from jax.experimental.pallas import tpu as pltpu
from jax.experimental.pallas import tpu_sc as plsc

N = 100000
E = 3200000
D = 16

NC = 2
NS = 16
NW = NC * NS

EPT = 102400
ROWS_PER_TILE = EPT // 128
E_PAD = EPT * NW
REAL_ROWS = E // 128
PAD_ROWS = E_PAD // 128 - REAL_ROWS

NPAD = 100352
RPT = NPAD // NS
PAD_SINK_ROWS = 352


def _sc_aggregate(with_cnt: bool, CH: int):
    NITER = ROWS_PER_TILE // CH
    NQ = NITER // 4
    mesh = plsc.VectorSubcoreMesh(core_axis_name="c", subcore_axis_name="s")
    out_type = [jax.ShapeDtypeStruct((NC, NPAD, D), jnp.float32)]
    if with_cnt:
        out_type.append(jax.ShapeDtypeStruct((NC, NPAD), jnp.float32))
    scratch = [
        pltpu.VMEM_SHARED((NPAD, D), jnp.float32),
    ]
    scratch += [pltpu.VMEM((CH, 128), jnp.int32) for _ in range(8)]
    scratch += [pltpu.VMEM((CH * 128, D), jnp.float32) for _ in range(2)]
    scratch += [pltpu.SemaphoreType.DMA for _ in range(4)]
    scratch += [pltpu.SemaphoreType.DMA for _ in range(2)]
    scratch += [pltpu.SemaphoreType.DMA for _ in range(2)]
    if with_cnt:
        scratch += [
            pltpu.VMEM_SHARED((NPAD,), jnp.float32),
            pltpu.VMEM((CH * 128,), jnp.float32),
            pltpu.SemaphoreType.DMA,
            pltpu.SemaphoreType.DMA,
        ]

    def body(x_hbm, srcR, dstR, srcP, dstP, z16_hbm, z1_hbm, *rest):
        if with_cnt:
            (agg_out, cnt_out, agg_sp,
             sv0, sv1, sv2, sv3, dv0, dv1, dv2, dv3, rv0, rv1,
             is0, is1, is2, is3, gs0, gs1, ss0, ss1,
             cnt_sp, ones_v, cs0, cs1) = rest
        else:
            (agg_out, agg_sp,
             sv0, sv1, sv2, sv3, dv0, dv1, dv2, dv3, rv0, rv1,
             is0, is1, is2, is3, gs0, gs1, ss0, ss1) = rest
        sv = [sv0, sv1, sv2, sv3]
        dv = [dv0, dv1, dv2, dv3]
        rv = [rv0, rv1]
        isem = [is0, is1, is2, is3]
        gsem = [gs0, gs1]
        ssem = [ss0, ss1]
        if with_cnt:
            csem = [cs0, cs1]
        c = lax.axis_index("c")
        s = lax.axis_index("s")
        wid = s * NC + c

        pltpu.sync_copy(z16_hbm.at[pl.ds(s * RPT, RPT)],
                        agg_sp.at[pl.ds(s * RPT, RPT)])
        if with_cnt:
            pltpu.sync_copy(z1_hbm.at[pl.ds(s * RPT, RPT)],
                            cnt_sp.at[pl.ds(s * RPT, RPT)])
            for i in range(CH * 8):
                ones_v[pl.ds(i * 16, 16)] = jnp.full((16,), 1.0, jnp.float32)
        plsc.subcore_barrier()

        row0 = wid * ROWS_PER_TILE

        def fire_idx(r, k):
            @pl.when(r < REAL_ROWS)
            def _():
                pltpu.async_copy(srcR.at[pl.ds(r, CH)], sv[k], isem[k])
                pltpu.async_copy(dstR.at[pl.ds(r, CH)], dv[k], isem[k])

            @pl.when(r >= REAL_ROWS)
            def _():
                rp = r - REAL_ROWS
                pltpu.async_copy(srcP.at[pl.ds(rp, CH)], sv[k], isem[k])
                pltpu.async_copy(dstP.at[pl.ds(rp, CH)], dv[k], isem[k])

        def wait_idx(k):
            pltpu.make_async_copy(srcR.at[pl.ds(0, CH)], sv[k],
                                  isem[k]).wait()
            pltpu.make_async_copy(dstR.at[pl.ds(0, CH)], dv[k],
                                  isem[k]).wait()

        def fire_gathers(k, m):
            for j in range(CH):
                pltpu.async_copy(x_hbm.at[sv[k].at[j]],
                                 rv[m].at[pl.ds(j * 128, 128)], gsem[m])

        def wait_gathers(m):
            pltpu.make_async_copy(x_hbm.at[pl.ds(0, CH * 128)], rv[m],
                                  gsem[m]).wait()

        def fire_scatters(k, m):
            for j in range(CH):
                pltpu.async_copy(rv[m].at[pl.ds(j * 128, 128)],
                                 agg_sp.at[dv[k].at[j]], ssem[m], add=True)
            if with_cnt:
                for j in range(CH):
                    pltpu.async_copy(ones_v.at[pl.ds(j * 128, 128)],
                                     cnt_sp.at[dv[k].at[j]], csem[m],
                                     add=True)

        def wait_scatters(m):
            pltpu.make_async_copy(rv[m], agg_sp.at[pl.ds(0, CH * 128)],
                                  ssem[m]).wait()
            if with_cnt:
                pltpu.make_async_copy(ones_v, cnt_sp.at[pl.ds(0, CH * 128)],
                                      csem[m]).wait()

        pltpu.sync_copy(srcR.at[pl.ds(row0, CH)], sv[0])
        pltpu.sync_copy(dstR.at[pl.ds(row0, CH)], dv[0])
        fire_gathers(0, 0)
        pltpu.async_copy(srcR.at[pl.ds(row0 + CH, CH)], sv[1], isem[1])
        pltpu.async_copy(dstR.at[pl.ds(row0 + CH, CH)], dv[1], isem[1])

        def quad(q, carry):
            r0 = row0 + 4 * q * CH
            for k in range(4):
                m = k % 2
                wait_gathers(m)

                if k < 2:
                    fire_idx(r0 + (k + 2) * CH, (k + 2) % 4)
                else:
                    @pl.when(q < NQ - 1)
                    def _(k=k, r0=r0):
                        fire_idx(r0 + (k + 2) * CH, (k + 2) % 4)

                if k == 0:
                    @pl.when(q > 0)
                    def _():
                        wait_scatters(1)
                else:
                    wait_scatters((k - 1) % 2)

                fire_scatters(k, m)

                if k < 3:
                    wait_idx(k + 1)
                    fire_gathers(k + 1, (k + 1) % 2)
                else:
                    @pl.when(q < NQ - 1)
                    def _():
                        wait_idx(0)
                        fire_gathers(0, 0)
            return carry

        lax.fori_loop(0, NQ, quad, 0)
        wait_scatters(1)
        plsc.subcore_barrier()

        pltpu.sync_copy(agg_sp.at[pl.ds(s * RPT, RPT)],
                        agg_out.at[c, pl.ds(s * RPT, RPT)])
        if with_cnt:
            pltpu.sync_copy(cnt_sp.at[pl.ds(s * RPT, RPT)],
                            cnt_out.at[c, pl.ds(s * RPT, RPT)])

    return pl.kernel(body, out_type=tuple(out_type), mesh=mesh,
                     scratch_types=scratch,
                     compiler_params=pltpu.CompilerParams(
                         use_tc_tiling_on_sc=False))


_sc_pass1 = _sc_aggregate(with_cnt=True, CH=4)
_sc_pass2 = _sc_aggregate(with_cnt=False, CH=5)


NP8 = NPAD // 8
NR = N // 8
BLKC = 1792
BLKR = 1792


def _sage_body(agg_a, agg_b, cnt_a, cnt_b, rep, x, wl, bl, wr, o):
    inv = 1.0 / jnp.maximum(cnt_a[0] + cnt_b[0], 1.0)
    inv16 = jnp.dot(inv, rep[...], preferred_element_type=jnp.float32)
    mean = (agg_a[0] + agg_b[0]) * inv16
    y = (jnp.dot(mean, wl[...], preferred_element_type=jnp.float32)
         + jnp.dot(x[...], wr[...], preferred_element_type=jnp.float32)
         + bl[...])
    o[...] = jnp.maximum(y, 0.0)


_sage_tc = pl.pallas_call(
    _sage_body,
    grid=(NP8 // BLKR,),
    in_specs=[pl.BlockSpec((1, BLKR, 128), lambda i: (0, i, 0)),
              pl.BlockSpec((1, BLKR, 128), lambda i: (1, i, 0)),
              pl.BlockSpec((1, BLKR, 8), lambda i: (0, i, 0)),
              pl.BlockSpec((1, BLKR, 8), lambda i: (1, i, 0)),
              pl.BlockSpec((8, 128), lambda i: (0, 0)),
              pl.BlockSpec((BLKR, 128), lambda i: (i, 0)),
              pl.BlockSpec((128, 128), lambda i: (0, 0)),
              pl.BlockSpec((1, 128), lambda i: (0, 0)),
              pl.BlockSpec((128, 128), lambda i: (0, 0))],
    out_specs=pl.BlockSpec((BLKR, 128), lambda i: (i, 0)),
    out_shape=jax.ShapeDtypeStruct((NR, 128), jnp.float32),
)


def _final_body(agg_a, agg_b, cnt_a, cnt_b, rep, x1, x0,
                wl, bl, wr, w3a, w3b, w3c, b3, o):
    inv = 1.0 / jnp.maximum(cnt_a[0] + cnt_b[0], 1.0)
    inv16 = jnp.dot(inv, rep[...], preferred_element_type=jnp.float32)
    mean = (agg_a[0] + agg_b[0]) * inv16
    y = (jnp.dot(mean, wl[...], preferred_element_type=jnp.float32)
         + jnp.dot(x1[...], wr[...], preferred_element_type=jnp.float32)
         + bl[...])
    x2 = jnp.maximum(y, 0.0)
    o[...] = (jnp.dot(x0[...], w3a[...], preferred_element_type=jnp.float32)
              + jnp.dot(x1[...], w3b[...], preferred_element_type=jnp.float32)
              + jnp.dot(x2, w3c[...], preferred_element_type=jnp.float32)
              + b3[...])


_final_tc = pl.pallas_call(
    _final_body,
    grid=(NP8 // BLKR,),
    in_specs=[pl.BlockSpec((1, BLKR, 128), lambda i: (0, i, 0)),
              pl.BlockSpec((1, BLKR, 128), lambda i: (1, i, 0)),
              pl.BlockSpec((1, BLKR, 8), lambda i: (0, i, 0)),
              pl.BlockSpec((1, BLKR, 8), lambda i: (1, i, 0)),
              pl.BlockSpec((8, 128), lambda i: (0, 0)),
              pl.BlockSpec((BLKR, 128), lambda i: (i, 0)),
              pl.BlockSpec((BLKR, 128), lambda i: (i, 0)),
              pl.BlockSpec((128, 128), lambda i: (0, 0)),
              pl.BlockSpec((1, 128), lambda i: (0, 0)),
              pl.BlockSpec((128, 128), lambda i: (0, 0)),
              pl.BlockSpec((128, 256), lambda i: (0, 0)),
              pl.BlockSpec((128, 256), lambda i: (0, 0)),
              pl.BlockSpec((128, 256), lambda i: (0, 0)),
              pl.BlockSpec((1, 256), lambda i: (0, 0))],
    out_specs=pl.BlockSpec((BLKR, 256), lambda i: (i, 0)),
    out_shape=jax.ShapeDtypeStruct((NR, 256), jnp.float32),
)


def kernel(x0, edge_index, Wl1, bl1, Wr1, Wl2, bl2, Wr2, W3, b3):
    src = edge_index[0].astype(jnp.int32)
    dst = edge_index[1].astype(jnp.int32)

    npad_e = E_PAD - E
    pad_idx = jnp.arange(npad_e, dtype=jnp.int32)
    pad_src = ((pad_idx * 7919) % N).reshape(PAD_ROWS, 128)
    pad_dst = (N + (pad_idx % PAD_SINK_ROWS)).reshape(PAD_ROWS, 128)
    src2d = src.reshape(REAL_ROWS, 128)
    dst2d = dst.reshape(REAL_ROWS, 128)

    z16 = jnp.zeros((NPAD, D), jnp.float32)
    z1 = jnp.zeros((NPAD,), jnp.float32)

    eye8 = jnp.eye(8, dtype=jnp.float32)
    rep = jnp.kron(eye8, jnp.ones((1, 16), jnp.float32))
    wl1bd = jnp.kron(eye8, Wl1)
    wr1bd = jnp.kron(eye8, Wr1)
    wl2bd = jnp.kron(eye8, Wl2)
    wr2bd = jnp.kron(eye8, Wr2)
    w3abd = jnp.kron(eye8, W3[0:16])
    w3bbd = jnp.kron(eye8, W3[16:32])
    w3cbd = jnp.kron(eye8, W3[32:48])
    bl1t = jnp.tile(bl1, 8).reshape(1, 128)
    bl2t = jnp.tile(bl2, 8).reshape(1, 128)
    b3t = jnp.tile(b3, 8).reshape(1, 256)

    agg1, cnt = _sc_pass1(x0, src2d, dst2d, pad_src, pad_dst, z16, z1)
    cntp = cnt.reshape(NC, NP8, 8)

    x0p = x0.reshape(NR, 128)
    x1p = _sage_tc(agg1.reshape(NC, NP8, 128), agg1.reshape(NC, NP8, 128),
                   cntp, cntp, rep, x0p, wl1bd, bl1t, wr1bd)

    (agg2,) = _sc_pass2(x1p.reshape(N, D), src2d, dst2d, pad_src, pad_dst, z16, z1)

    outp = _final_tc(agg2.reshape(NC, NP8, 128), agg2.reshape(NC, NP8, 128),
                     cntp, cntp, rep, x1p, x0p, wl2bd, bl2t, wr2bd,
                     w3abd, w3bbd, w3cbd, b3t)
    return outp.reshape(N, 32)
